# baseline (speedup 1.0000x reference)
import jax
import jax.numpy as jnp
from jax import lax
from jax.experimental import pallas as pl
from jax.experimental.pallas import tpu as pltpu

VOCAB = 100000
HIDDEN = 128
N_TOK = 512

_NV = 10
_VT = 10240
_TAIL = VOCAB - (_NV - 1) * _VT
_REST = _VT - _TAIL


def _w_copies(w_ref, w_buf, sem_w, v, slot):
    head = pltpu.make_async_copy(
        w_ref.at[pl.ds(v * _VT, _TAIL), :],
        w_buf.at[slot, pl.ds(0, _TAIL), :],
        sem_w.at[slot],
    )
    rest = pltpu.make_async_copy(
        w_ref.at[pl.ds(v * _VT + _TAIL, _REST), :],
        w_buf.at[slot, pl.ds(_TAIL, _REST), :],
        sem_w.at[slot],
    )
    return head, rest


def _start_w(w_ref, w_buf, sem_w, v, slot):
    head, rest = _w_copies(w_ref, w_buf, sem_w, v, slot)
    head.start()

    @pl.when(v < _NV - 1)
    def _():
        rest.start()


def _wait_w(w_ref, w_buf, sem_w, v, slot):
    head, rest = _w_copies(w_ref, w_buf, sem_w, v, slot)
    head.wait()

    @pl.when(v < _NV - 1)
    def _():
        rest.wait()


def _body(idx_ref, we_ref, w_ref, b_ref, o_ref, h_raw, h_bf, w_buf, sem_h, sem_w):
    v = pl.program_id(0)
    slot = lax.rem(v, 2)

    @pl.when(v == 0)
    def _prologue():
        def issue(i, _):
            pltpu.make_async_copy(
                we_ref.at[pl.ds(idx_ref[i], 1), :], h_raw.at[pl.ds(i, 1), :], sem_h
            ).start()
            return 0

        lax.fori_loop(0, N_TOK, issue, 0, unroll=16)
        _start_w(w_ref, w_buf, sem_w, 0, 0)
        pltpu.make_async_copy(we_ref.at[pl.ds(0, N_TOK), :], h_raw, sem_h).wait()
        h_bf[...] = h_raw[...].astype(jnp.bfloat16)

    @pl.when(v < _NV - 1)
    def _prefetch():
        _start_w(w_ref, w_buf, sem_w, v + 1, 1 - slot)

    _wait_w(w_ref, w_buf, sem_w, v, slot)
    acc = lax.dot_general(
        h_bf[...],
        w_buf[slot].astype(jnp.bfloat16),
        (((1,), (1,)), ((), ())),
        preferred_element_type=jnp.float32,
    )
    o_ref[...] = acc + b_ref[...]


def kernel(x, we, W, b):
    bsz, seq = x.shape
    idx = x.reshape(N_TOK).astype(jnp.int32)
    out = pl.pallas_call(
        _body,
        grid_spec=pltpu.PrefetchScalarGridSpec(
            num_scalar_prefetch=1,
            grid=(_NV,),
            in_specs=[
                pl.BlockSpec(memory_space=pltpu.HBM),
                pl.BlockSpec(memory_space=pltpu.HBM),
                pl.BlockSpec((1, _VT), lambda v, idx: (0, v)),
            ],
            out_specs=pl.BlockSpec((N_TOK, _VT), lambda v, idx: (0, v)),
            scratch_shapes=[
                pltpu.VMEM((N_TOK, HIDDEN), jnp.float32),
                pltpu.VMEM((N_TOK, HIDDEN), jnp.bfloat16),
                pltpu.VMEM((2, _VT, HIDDEN), jnp.float32),
                pltpu.SemaphoreType.DMA,
                pltpu.SemaphoreType.DMA((2,)),
            ],
        ),
        out_shape=jax.ShapeDtypeStruct((N_TOK, VOCAB), jnp.float32),
        compiler_params=pltpu.CompilerParams(
            dimension_semantics=("arbitrary",),
        ),
    )(idx, we, W, b.reshape(1, VOCAB))
    return out.reshape(bsz, seq, VOCAB)


# speedup vs baseline: 1.0132x; 1.0132x over previous
import jax
import jax.numpy as jnp
from jax import lax
from jax.experimental import pallas as pl
from jax.experimental.pallas import tpu as pltpu

VOCAB = 100000
HIDDEN = 128
N_TOK = 512

_VT = 10240


def _body(idx_ref, we_ref, w_ref, b_ref, o_ref, h_raw, h_bf, sem):
    v = pl.program_id(0)

    @pl.when(v == 0)
    def _gather():
        def issue(i, _):
            pltpu.make_async_copy(
                we_ref.at[pl.ds(idx_ref[i], 1), :], h_raw.at[pl.ds(i, 1), :], sem
            ).start()
            return 0

        lax.fori_loop(0, N_TOK, issue, 0, unroll=32)
        pltpu.make_async_copy(
            we_ref.at[pl.ds(0, N_TOK), :], h_raw, sem
        ).wait()
        h_bf[...] = h_raw[...].astype(jnp.bfloat16)

    w = w_ref[...].astype(jnp.bfloat16)
    acc = lax.dot_general(
        h_bf[...], w, (((1,), (1,)), ((), ())), preferred_element_type=jnp.float32
    )
    o_ref[...] = acc + b_ref[...]


def kernel(x, we, W, b):
    bsz, seq = x.shape
    idx = x.reshape(N_TOK).astype(jnp.int32)
    grid = (pl.cdiv(VOCAB, _VT),)
    out = pl.pallas_call(
        _body,
        grid_spec=pltpu.PrefetchScalarGridSpec(
            num_scalar_prefetch=1,
            grid=grid,
            in_specs=[
                pl.BlockSpec(memory_space=pltpu.HBM),
                pl.BlockSpec((_VT, HIDDEN), lambda v, idx: (v, 0)),
                pl.BlockSpec((1, _VT), lambda v, idx: (0, v)),
            ],
            out_specs=pl.BlockSpec((N_TOK, _VT), lambda v, idx: (0, v)),
            scratch_shapes=[
                pltpu.VMEM((N_TOK, HIDDEN), jnp.float32),
                pltpu.VMEM((N_TOK, HIDDEN), jnp.bfloat16),
                pltpu.SemaphoreType.DMA,
            ],
        ),
        out_shape=jax.ShapeDtypeStruct((N_TOK, VOCAB), jnp.float32),
        compiler_params=pltpu.CompilerParams(
            dimension_semantics=("arbitrary",),
        ),
    )(idx, we, W, b.reshape(1, VOCAB))
    return out.reshape(bsz, seq, VOCAB)
